# Initial kernel scaffold; baseline (speedup 1.0000x reference)
import jax
import jax.numpy as jnp
from jax.experimental import pallas as pl


def kernel(x, delta_t_info, delta_dis_info, delta_rg_info, delta_entropy_info, city_embeddings, router_w, router_b, c_fc_w, c_fc_b, c_proj_w, c_proj_b, city):
    raise NotImplementedError("write your pallas kernel here")


# speedup vs baseline: 1.0891x; 1.0891x over previous
import functools

import jax
import jax.numpy as jnp
from jax.experimental import pallas as pl
from jax.experimental.pallas import tpu as pltpu

_N_EMBD = 768
_NE = 8
_BT = 256
_C1 = 0.7978845608028654


def _gelu(x):
    return 0.5 * x * (1.0 + jnp.tanh(_C1 * (x + 0.044715 * x * x * x)))


def _moe_body(x_ref, g_ref, w1_ref, b1_ref, w2_ref, b2_ref, out_ref):
    e = pl.program_id(0)
    t = pl.program_id(1)
    xb = x_ref[...]
    w1 = w1_ref[0]
    h = jax.lax.dot_general(xb, w1, (((1,), (1,)), ((), ())),
                            preferred_element_type=jnp.float32)
    h = _gelu(h + b1_ref[0])
    w2 = w2_ref[0]
    o = jax.lax.dot_general(h, w2, (((1,), (1,)), ((), ())),
                            preferred_element_type=jnp.float32)
    o = o + b2_ref[0]
    g2d = g_ref[...]
    lane = jax.lax.broadcasted_iota(jnp.int32, (_BT, _NE), 1)
    gcol = jnp.sum(jnp.where(lane == e, g2d, 0.0), axis=1, keepdims=True)
    o = o * gcol

    @pl.when(e == 0)
    def _():
        out_ref[pl.ds(t * _BT, _BT), :] = o

    @pl.when(e != 0)
    def _():
        out_ref[pl.ds(t * _BT, _BT), :] += o


def kernel(x, delta_t_info, delta_dis_info, delta_rg_info, delta_entropy_info,
           city_embeddings, router_w, router_b, c_fc_w, c_fc_b, c_proj_w, c_proj_b, city):
    B, T, C = x.shape
    ce = city_embeddings[city]
    ce_b = jnp.broadcast_to(ce[None, None, :], (B, T, 32))
    h = jnp.concatenate([x, ce_b, delta_t_info, delta_dis_info,
                         delta_rg_info, delta_entropy_info], axis=-1)
    logits = jnp.einsum('btl,el->bte', h, router_w) + router_b
    gate1 = jax.nn.softmax(logits, axis=-1)
    top_vals, top_idx = jax.lax.top_k(logits, 2)
    sel_mask = jnp.sum(jax.nn.one_hot(top_idx, _NE, dtype=jnp.float32), axis=-2) > 0
    sparse_logits = jnp.where(sel_mask, logits, -jnp.inf)
    router_output = jax.nn.softmax(sparse_logits, axis=-1)

    flat_x = x.reshape(T, C)
    flat_g = router_output.reshape(T, _NE)

    out = pl.pallas_call(
        _moe_body,
        grid=(_NE, T // _BT),
        in_specs=[
            pl.BlockSpec((_BT, C), lambda e, t: (t, 0)),
            pl.BlockSpec((_BT, _NE), lambda e, t: (t, 0)),
            pl.BlockSpec((1, 4 * C, C), lambda e, t: (e, 0, 0)),
            pl.BlockSpec((1, 1, 4 * C), lambda e, t: (e, 0, 0)),
            pl.BlockSpec((1, C, 4 * C), lambda e, t: (e, 0, 0)),
            pl.BlockSpec((1, 1, C), lambda e, t: (e, 0, 0)),
        ],
        out_specs=pl.BlockSpec((T, C), lambda e, t: (0, 0)),
        out_shape=jax.ShapeDtypeStruct((T, C), jnp.float32),
    )(flat_x, flat_g, c_fc_w, c_fc_b.reshape(_NE, 1, 4 * C),
      c_proj_w, c_proj_b.reshape(_NE, 1, C))

    return out.reshape(B, T, C), gate1
